# baseline (speedup 1.0000x reference)
import jax
import jax.numpy as jnp
from jax import lax
from jax.experimental import pallas as pl
from jax.experimental.pallas import tpu as pltpu
from jax.experimental.pallas import tpu_sc as plsc

B = 4; IN_CH = 80; T = 512; D = 256; DFF = 1024; G = 4; EPG = 4; E = 16
H = 4; DH = D // H; L = 2; S = 512; TOPK = 2; NCLS = 35
N = B * S
TILE = 128
P = N * TOPK + E * TILE
NT = P // TILE
KIN = 3 * IN_CH


def _gelu(v):
    return 0.5 * v * (1.0 + lax.erf(v * 0.7071067811865476))


def _layernorm(x, g, b):
    m = jnp.mean(x, axis=-1, keepdims=True)
    v = jnp.mean((x - m) ** 2, axis=-1, keepdims=True)
    return (x - m) * lax.rsqrt(v + 1e-5) * g + b


def _stem_body(x_ref, w_ref, b_ref, pos_ref, o_ref):
    y = jnp.dot(x_ref[...], w_ref[...], preferred_element_type=jnp.float32)
    o_ref[...] = _gelu(y + b_ref[...]) + pos_ref[...]


def _attn_body(h_ref, g_ref, bb_ref, wq_ref, bq_ref, wk_ref, bk_ref,
               wv_ref, bv_ref, wo_ref, bo_ref, o_ref):
    hh = h_ref[...]
    hn = _layernorm(hh, g_ref[...], bb_ref[...])
    q = jnp.dot(hn, wq_ref[...], preferred_element_type=jnp.float32) + bq_ref[...]
    k = jnp.dot(hn, wk_ref[...], preferred_element_type=jnp.float32) + bk_ref[...]
    v = jnp.dot(hn, wv_ref[...], preferred_element_type=jnp.float32) + bv_ref[...]
    s = lax.dot_general(q, k, (((1,), (1,)), ((), ())),
                        preferred_element_type=jnp.float32) * (DH ** -0.5)
    s = s - jnp.max(s, axis=-1, keepdims=True)
    e = jnp.exp(s)
    att = e / jnp.sum(e, axis=-1, keepdims=True)
    oh = jnp.dot(att, v, preferred_element_type=jnp.float32)
    part = jnp.dot(oh, wo_ref[...], preferred_element_type=jnp.float32)

    @pl.when(pl.program_id(1) == 0)
    def _init():
        o_ref[...] = hh + part + bo_ref[...]

    @pl.when(pl.program_id(1) != 0)
    def _acc():
        o_ref[...] += part


def _router_body(h_ref, g_ref, b_ref, wg_ref, bg_ref, we_ref, be_ref,
                 hn_ref, meta_ref):
    hn = _layernorm(h_ref[...], g_ref[...], b_ref[...])
    hn_ref[...] = hn.astype(jnp.bfloat16)
    gl = jnp.dot(hn, wg_ref[...], preferred_element_type=jnp.float32) + bg_ref[...]
    el = jnp.dot(hn, we_ref[...], preferred_element_type=jnp.float32) + be_ref[...]
    lane = lax.broadcasted_iota(jnp.int32, gl.shape, 1)
    neg = jnp.float32(-1e30)
    glm = jnp.where(lane < G, gl, neg)
    ge = jnp.where(lane < G, jnp.exp(glm - jnp.max(glm, -1, keepdims=True)), 0.0)
    gp = ge / jnp.sum(ge, -1, keepdims=True)
    elm = jnp.where(lane < E, el, neg)
    ee = jnp.where(lane < E, jnp.exp(elm - jnp.max(elm, -1, keepdims=True)), 0.0)
    r = lax.broadcasted_iota(jnp.int32, (128, 128), 0)
    c = lax.broadcasted_iota(jnp.int32, (128, 128), 1)
    grpsum = jnp.where((r < E) & (c < E) & ((r // EPG) == (c // EPG)), 1.0, 0.0)
    es = jnp.dot(ee, grpsum, preferred_element_type=jnp.float32,
                 precision=lax.Precision.HIGHEST)
    ep = ee / jnp.where(lane < E, es, 1.0)
    bcast = jnp.where((r < G) & (c < E) & ((c // EPG) == r), 1.0, 0.0)
    gpb = jnp.dot(gp, bcast, preferred_element_type=jnp.float32,
                  precision=lax.Precision.HIGHEST)
    comb = jnp.where(lane < E, gpb * ep, neg)
    big = jnp.int32(1 << 30)
    v1 = jnp.max(comb, -1, keepdims=True)
    a1 = jnp.min(jnp.where(comb == v1, lane, big), -1, keepdims=True)
    comb2 = jnp.where(lane == a1, neg, comb)
    v2 = jnp.max(comb2, -1, keepdims=True)
    a2 = jnp.min(jnp.where(comb2 == v2, lane, big), -1, keepdims=True)
    ssum = v1 + v2 + 1e-9
    meta = jnp.where(lane == 0, a1.astype(jnp.float32), 0.0)
    meta = meta + jnp.where(lane == 1, a2.astype(jnp.float32), 0.0)
    meta = meta + jnp.where(lane == 2, v1 / ssum, 0.0)
    meta = meta + jnp.where(lane == 3, v2 / ssum, 0.0)
    meta_ref[...] = meta


def _sc_gather_rows(table, idx, nrows):
    v, dt = table.shape
    rpw = nrows // 32
    mesh = plsc.VectorSubcoreMesh(core_axis_name="c", subcore_axis_name="s")

    def body(table_hbm, idx_hbm, out_hbm, idx_v, rows_v, sem):
        wid = lax.axis_index("s") * 2 + lax.axis_index("c")
        base = wid * rpw
        pltpu.sync_copy(idx_hbm.at[pl.ds(base, rpw)], idx_v)
        pltpu.async_copy(table_hbm.at[idx_v], rows_v, sem).wait()
        pltpu.sync_copy(rows_v, out_hbm.at[pl.ds(base, rpw)])

    return pl.kernel(
        body, mesh=mesh,
        out_type=jax.ShapeDtypeStruct((nrows, dt), table.dtype),
        scratch_types=[
            pltpu.VMEM((rpw,), jnp.int32),
            pltpu.VMEM((rpw, dt), table.dtype),
            pltpu.SemaphoreType.DMA,
        ],
    )(table, idx)


def _ffn_body(te_ref, x_ref, w1_ref, b1_ref, w2_ref, b2_ref, o_ref):
    h1 = _gelu(jnp.dot(x_ref[...], w1_ref[...],
                       preferred_element_type=jnp.float32) + b1_ref[...])
    o_ref[...] = jnp.dot(h1.astype(jnp.bfloat16), w2_ref[...],
                         preferred_element_type=jnp.float32) + b2_ref[...]


def _head_body(h_ref, g_ref, b_ref, w_ref, bh_ref, o_ref):
    hh = h_ref[...]
    r = lax.broadcasted_iota(jnp.int32, (8, N), 0)
    c = lax.broadcasted_iota(jnp.int32, (8, N), 1)
    pool = jnp.where((c // S) == r, 1.0 / S, 0.0)
    p = jnp.dot(pool, hh, preferred_element_type=jnp.float32,
                precision=lax.Precision.HIGHEST)
    pn = _layernorm(p, g_ref[...], b_ref[...])
    o_ref[...] = jnp.dot(pn, w_ref[...],
                         preferred_element_type=jnp.float32) + bh_ref[...]


def kernel(x, conv_w, conv_b, pos_emb, ln1_g, ln1_b, wq, bq, wk, bk, wv, bv,
           wo, bo, ln2_g, ln2_b, gr_w, gr_b, er_w, er_b, w1, b1, w2, b2,
           lnf_g, lnf_b, head_w, head_b):
    f32 = jnp.float32

    xt = jnp.transpose(x, (0, 2, 1))
    xp = jnp.pad(xt, ((0, 0), (1, 1), (0, 0)))
    win = jnp.concatenate([xp[:, 0:T], xp[:, 1:T + 1], xp[:, 2:T + 2]], -1)
    xi = jnp.pad(win.reshape(N, KIN), ((0, 0), (0, 256 - KIN)))
    wmat = jnp.pad(conv_w.transpose(2, 1, 0).reshape(KIN, D),
                   ((0, 256 - KIN), (0, 0)))
    pos = jnp.tile(pos_emb[:S], (B, 1))
    h = pl.pallas_call(
        _stem_body,
        out_shape=jax.ShapeDtypeStruct((N, D), f32),
    )(xi, wmat, conv_b.reshape(1, D), pos)

    for l in range(L):
        h4 = h.reshape(B, S, D)
        wqh = wq[l].reshape(D, H, DH).transpose(1, 0, 2)
        wkh = wk[l].reshape(D, H, DH).transpose(1, 0, 2)
        wvh = wv[l].reshape(D, H, DH).transpose(1, 0, 2)
        woh = wo[l].reshape(H, DH, D)
        bqh = bq[l].reshape(H, 1, DH)
        bkh = bk[l].reshape(H, 1, DH)
        bvh = bv[l].reshape(H, 1, DH)
        h = pl.pallas_call(
            _attn_body,
            grid=(B, H),
            in_specs=[
                pl.BlockSpec((None, S, D), lambda b, hh: (b, 0, 0)),
                pl.BlockSpec((1, D), lambda b, hh: (0, 0)),
                pl.BlockSpec((1, D), lambda b, hh: (0, 0)),
                pl.BlockSpec((None, D, DH), lambda b, hh: (hh, 0, 0)),
                pl.BlockSpec((None, 1, DH), lambda b, hh: (hh, 0, 0)),
                pl.BlockSpec((None, D, DH), lambda b, hh: (hh, 0, 0)),
                pl.BlockSpec((None, 1, DH), lambda b, hh: (hh, 0, 0)),
                pl.BlockSpec((None, D, DH), lambda b, hh: (hh, 0, 0)),
                pl.BlockSpec((None, 1, DH), lambda b, hh: (hh, 0, 0)),
                pl.BlockSpec((None, DH, D), lambda b, hh: (hh, 0, 0)),
                pl.BlockSpec((1, D), lambda b, hh: (0, 0)),
            ],
            out_specs=pl.BlockSpec((None, S, D), lambda b, hh: (b, 0, 0)),
            out_shape=jax.ShapeDtypeStruct((B, S, D), f32),
        )(h4, ln1_g[l].reshape(1, D), ln1_b[l].reshape(1, D),
          wqh, bqh, wkh, bkh, wvh, bvh, woh, bo[l].reshape(1, D)).reshape(N, D)

        wg_pad = jnp.pad(gr_w[l], ((0, 0), (0, 128 - G)))
        bg_pad = jnp.pad(gr_b[l].reshape(1, G), ((0, 0), (0, 128 - G)))
        we_pad = jnp.pad(er_w[l], ((0, 0), (0, 128 - E)))
        be_pad = jnp.pad(er_b[l].reshape(1, E), ((0, 0), (0, 128 - E)))
        hn2, meta = pl.pallas_call(
            _router_body,
            out_shape=(jax.ShapeDtypeStruct((N, D), jnp.bfloat16),
                       jax.ShapeDtypeStruct((N, 128), f32)),
        )(h, ln2_g[l].reshape(1, D), ln2_b[l].reshape(1, D),
          wg_pad, bg_pad, we_pad, be_pad)

        e1 = meta[:, 0].astype(jnp.int32)
        e2 = meta[:, 1].astype(jnp.int32)
        g1 = meta[:, 2]
        g2 = meta[:, 3]
        flat_e = jnp.stack([e1, e2], 1).reshape(-1)
        oh = (flat_e[:, None] == jnp.arange(E)[None, :]).astype(jnp.int32)
        csum = jnp.cumsum(oh, 0)
        counts = csum[-1]
        rank = jnp.sum(csum * oh, 1) - 1
        pc = ((counts + TILE - 1) // TILE) * TILE
        zero1 = jnp.zeros((1,), jnp.int32)
        pstart = jnp.concatenate([zero1, jnp.cumsum(pc)[:-1]])
        dest = jnp.sum(oh * pstart[None, :], 1) + rank
        token_src = jnp.zeros((P,), jnp.int32).at[dest].set(
            jnp.arange(N * TOPK, dtype=jnp.int32) // TOPK)
        hn2_i32 = lax.bitcast_convert_type(hn2.reshape(N, 128, 2), jnp.int32)
        xg_i32 = _sc_gather_rows(hn2_i32, token_src, P)
        xg = lax.bitcast_convert_type(xg_i32, jnp.bfloat16).reshape(P, D)
        t_start = jnp.arange(NT, dtype=jnp.int32) * TILE
        te = jnp.clip(jnp.sum(t_start[:, None] >= pstart[None, :], 1) - 1,
                      0, E - 1).astype(jnp.int32)

        y = pl.pallas_call(
            _ffn_body,
            grid_spec=pltpu.PrefetchScalarGridSpec(
                num_scalar_prefetch=1,
                grid=(NT,),
                in_specs=[
                    pl.BlockSpec((TILE, D), lambda t, te_r: (t, 0)),
                    pl.BlockSpec((None, D, DFF), lambda t, te_r: (te_r[t], 0, 0)),
                    pl.BlockSpec((None, 1, DFF), lambda t, te_r: (te_r[t], 0, 0)),
                    pl.BlockSpec((None, DFF, D), lambda t, te_r: (te_r[t], 0, 0)),
                    pl.BlockSpec((None, 1, D), lambda t, te_r: (te_r[t], 0, 0)),
                ],
                out_specs=pl.BlockSpec((TILE, D), lambda t, te_r: (t, 0)),
            ),
            out_shape=jax.ShapeDtypeStruct((P, D), f32),
        )(te, xg, w1[l].astype(jnp.bfloat16), b1[l].reshape(E, 1, DFF),
          w2[l].astype(jnp.bfloat16), b2[l].reshape(E, 1, D))

        d2 = dest.reshape(N, TOPK)
        y1 = _sc_gather_rows(y, d2[:, 0], N)
        y2 = _sc_gather_rows(y, d2[:, 1], N)
        h = h + y1 * g1[:, None] + y2 * g2[:, None]

    hw_pad = jnp.pad(head_w, ((0, 0), (0, 128 - NCLS)))
    bh_pad = jnp.pad(head_b.reshape(1, NCLS), ((0, 0), (0, 128 - NCLS)))
    logits = pl.pallas_call(
        _head_body,
        out_shape=jax.ShapeDtypeStruct((8, 128), f32),
    )(h, lnf_g.reshape(1, D), lnf_b.reshape(1, D), hw_pad, bh_pad)
    return logits[:B, :NCLS]


# speedup vs baseline: 1.7566x; 1.7566x over previous
import jax
import jax.numpy as jnp
from jax import lax
from jax.experimental import pallas as pl
from jax.experimental.pallas import tpu as pltpu

B = 4; IN_CH = 80; T = 512; D = 256; DFF = 1024; G = 4; EPG = 4; E = 16
H = 4; DH = D // H; L = 2; S = 512; TOPK = 2; NCLS = 35
N = B * S
TILE = 128
P = N * TOPK + E * TILE
NT = P // TILE
KIN = 3 * IN_CH


def _gelu(v):
    return 0.5 * v * (1.0 + lax.erf(v * 0.7071067811865476))


def _layernorm(x, g, b):
    m = jnp.mean(x, axis=-1, keepdims=True)
    v = jnp.mean((x - m) ** 2, axis=-1, keepdims=True)
    return (x - m) * lax.rsqrt(v + 1e-5) * g + b


def _stem_body(x_ref, w_ref, b_ref, pos_ref, o_ref):
    y = jnp.dot(x_ref[...], w_ref[...], preferred_element_type=jnp.float32)
    o_ref[...] = _gelu(y + b_ref[...]) + pos_ref[...]


def _attn_body(h_ref, g_ref, bb_ref, wq_ref, bq_ref, wk_ref, bk_ref,
               wv_ref, bv_ref, wo_ref, bo_ref, o_ref):
    hh = h_ref[...]
    hn = _layernorm(hh, g_ref[...], bb_ref[...])
    q = jnp.dot(hn, wq_ref[...], preferred_element_type=jnp.float32) + bq_ref[...]
    k = jnp.dot(hn, wk_ref[...], preferred_element_type=jnp.float32) + bk_ref[...]
    v = jnp.dot(hn, wv_ref[...], preferred_element_type=jnp.float32) + bv_ref[...]
    s = lax.dot_general(q, k, (((1,), (1,)), ((), ())),
                        preferred_element_type=jnp.float32) * (DH ** -0.5)
    s = s - jnp.max(s, axis=-1, keepdims=True)
    e = jnp.exp(s)
    att = e / jnp.sum(e, axis=-1, keepdims=True)
    oh = jnp.dot(att, v, preferred_element_type=jnp.float32)
    part = jnp.dot(oh, wo_ref[...], preferred_element_type=jnp.float32)

    @pl.when(pl.program_id(1) == 0)
    def _init():
        o_ref[...] = hh + part + bo_ref[...]

    @pl.when(pl.program_id(1) != 0)
    def _acc():
        o_ref[...] += part


def _router_body(h_ref, g_ref, b_ref, wg_ref, bg_ref, we_ref, be_ref,
                 hn_ref, meta_ref, te_ref):
    hn = _layernorm(h_ref[...], g_ref[...], b_ref[...])
    hn_ref[...] = hn.astype(jnp.bfloat16)
    gl = jnp.dot(hn, wg_ref[...], preferred_element_type=jnp.float32) + bg_ref[...]
    el = jnp.dot(hn, we_ref[...], preferred_element_type=jnp.float32) + be_ref[...]
    lane = lax.broadcasted_iota(jnp.int32, gl.shape, 1)
    neg = jnp.float32(-1e30)
    glm = jnp.where(lane < G, gl, neg)
    ge = jnp.where(lane < G, jnp.exp(glm - jnp.max(glm, -1, keepdims=True)), 0.0)
    gp = ge / jnp.sum(ge, -1, keepdims=True)
    elm = jnp.where(lane < E, el, neg)
    ee = jnp.where(lane < E, jnp.exp(elm - jnp.max(elm, -1, keepdims=True)), 0.0)
    r = lax.broadcasted_iota(jnp.int32, (128, 128), 0)
    c = lax.broadcasted_iota(jnp.int32, (128, 128), 1)
    grpsum = jnp.where((r < E) & (c < E) & ((r // EPG) == (c // EPG)), 1.0, 0.0)
    es = jnp.dot(ee, grpsum, preferred_element_type=jnp.float32,
                 precision=lax.Precision.HIGHEST)
    ep = ee / jnp.where(lane < E, es, 1.0)
    bcast = jnp.where((r < G) & (c < E) & ((c // EPG) == r), 1.0, 0.0)
    gpb = jnp.dot(gp, bcast, preferred_element_type=jnp.float32,
                  precision=lax.Precision.HIGHEST)
    comb = jnp.where(lane < E, gpb * ep, neg)
    big = jnp.int32(1 << 30)
    v1 = jnp.max(comb, -1, keepdims=True)
    a1 = jnp.min(jnp.where(comb == v1, lane, big), -1, keepdims=True)
    comb2 = jnp.where(lane == a1, neg, comb)
    v2 = jnp.max(comb2, -1, keepdims=True)
    a2 = jnp.min(jnp.where(comb2 == v2, lane, big), -1, keepdims=True)
    ssum = v1 + v2 + 1e-9
    oh1 = jnp.where((lane < E) & (lane == a1), 1.0, 0.0)
    oh2 = jnp.where((lane < E) & (lane == a2), 1.0, 0.0)
    u = oh1 + oh2
    tril = jnp.where(r > c, 1.0, 0.0)
    blocks = []
    carry = jnp.zeros((1, 128), jnp.float32)
    for cb in range(N // 128):
        blk = u[cb * 128:(cb + 1) * 128, :]
        blocks.append(jnp.dot(tril, blk, preferred_element_type=jnp.float32,
                              precision=lax.Precision.HIGHEST) + carry)
        carry = carry + jnp.sum(blk, 0, keepdims=True)
    uex = jnp.concatenate(blocks, axis=0)
    counts = carry
    pc = jnp.floor((counts + (TILE - 1)) * (1.0 / TILE)) * TILE
    sup = jnp.where((r < c) & (c < E), 1.0, 0.0)
    pstart = jnp.dot(pc, sup, preferred_element_type=jnp.float32,
                     precision=lax.Precision.HIGHEST)
    d1 = jnp.sum(oh1 * (pstart + uex), -1, keepdims=True)
    d2 = jnp.sum(oh2 * (pstart + uex + oh1), -1, keepdims=True)
    ts = (lax.broadcasted_iota(jnp.int32, (8, 128), 1) * TILE).astype(jnp.float32)
    acc = jnp.zeros((8, 128), jnp.float32)
    for e in range(E):
        acc = acc + jnp.where(ts >= pstart[0:1, e:e + 1], 1.0, 0.0)
    te_ref[...] = jnp.clip(acc - 1.0, 0.0, E - 1).astype(jnp.int32)
    meta = jnp.where(lane == 0, d1, 0.0)
    meta = meta + jnp.where(lane == 1, d2, 0.0)
    meta = meta + jnp.where(lane == 2, v1 / ssum, 0.0)
    meta = meta + jnp.where(lane == 3, v2 / ssum, 0.0)
    meta_ref[...] = meta


def _ffn_body(te_ref, x_ref, w1_ref, b1_ref, w2_ref, b2_ref, o_ref):
    h1 = _gelu(jnp.dot(x_ref[...], w1_ref[...],
                       preferred_element_type=jnp.float32) + b1_ref[...])
    o_ref[...] = jnp.dot(h1.astype(jnp.bfloat16), w2_ref[...],
                         preferred_element_type=jnp.float32) + b2_ref[...]


def _head_body(h_ref, wb_ref, y_ref, g_ref, b_ref, w_ref, bh_ref, o_ref):
    hh = h_ref[...]
    r = lax.broadcasted_iota(jnp.int32, (8, N), 0)
    c = lax.broadcasted_iota(jnp.int32, (8, N), 1)
    pool = jnp.where((c // S) == r, 1.0 / S, 0.0)
    p = jnp.dot(pool, hh, preferred_element_type=jnp.float32,
                precision=lax.Precision.HIGHEST)
    p = p + jnp.dot(wb_ref[...], y_ref[...], preferred_element_type=jnp.float32,
                    precision=lax.Precision.HIGHEST)
    pn = _layernorm(p, g_ref[...], b_ref[...])
    o_ref[...] = jnp.dot(pn, w_ref[...],
                         preferred_element_type=jnp.float32) + bh_ref[...]


def kernel(x, conv_w, conv_b, pos_emb, ln1_g, ln1_b, wq, bq, wk, bk, wv, bv,
           wo, bo, ln2_g, ln2_b, gr_w, gr_b, er_w, er_b, w1, b1, w2, b2,
           lnf_g, lnf_b, head_w, head_b):
    f32 = jnp.float32

    xt = jnp.transpose(x, (0, 2, 1))
    xp = jnp.pad(xt, ((0, 0), (1, 1), (0, 0)))
    win = jnp.concatenate([xp[:, 0:T], xp[:, 1:T + 1], xp[:, 2:T + 2]], -1)
    xi = jnp.pad(win.reshape(N, KIN), ((0, 0), (0, 256 - KIN)))
    wmat = jnp.pad(conv_w.transpose(2, 1, 0).reshape(KIN, D),
                   ((0, 256 - KIN), (0, 0)))
    pos = jnp.tile(pos_emb[:S], (B, 1))
    h = pl.pallas_call(
        _stem_body,
        out_shape=jax.ShapeDtypeStruct((N, D), f32),
    )(xi, wmat, conv_b.reshape(1, D), pos)

    for l in range(L):
        h4 = h.reshape(B, S, D)
        wqh = wq[l].reshape(D, H, DH).transpose(1, 0, 2)
        wkh = wk[l].reshape(D, H, DH).transpose(1, 0, 2)
        wvh = wv[l].reshape(D, H, DH).transpose(1, 0, 2)
        woh = wo[l].reshape(H, DH, D)
        bqh = bq[l].reshape(H, 1, DH)
        bkh = bk[l].reshape(H, 1, DH)
        bvh = bv[l].reshape(H, 1, DH)
        h = pl.pallas_call(
            _attn_body,
            grid=(B, H),
            in_specs=[
                pl.BlockSpec((None, S, D), lambda b, hh: (b, 0, 0)),
                pl.BlockSpec((1, D), lambda b, hh: (0, 0)),
                pl.BlockSpec((1, D), lambda b, hh: (0, 0)),
                pl.BlockSpec((None, D, DH), lambda b, hh: (hh, 0, 0)),
                pl.BlockSpec((None, 1, DH), lambda b, hh: (hh, 0, 0)),
                pl.BlockSpec((None, D, DH), lambda b, hh: (hh, 0, 0)),
                pl.BlockSpec((None, 1, DH), lambda b, hh: (hh, 0, 0)),
                pl.BlockSpec((None, D, DH), lambda b, hh: (hh, 0, 0)),
                pl.BlockSpec((None, 1, DH), lambda b, hh: (hh, 0, 0)),
                pl.BlockSpec((None, DH, D), lambda b, hh: (hh, 0, 0)),
                pl.BlockSpec((1, D), lambda b, hh: (0, 0)),
            ],
            out_specs=pl.BlockSpec((None, S, D), lambda b, hh: (b, 0, 0)),
            out_shape=jax.ShapeDtypeStruct((B, S, D), f32),
        )(h4, ln1_g[l].reshape(1, D), ln1_b[l].reshape(1, D),
          wqh, bqh, wkh, bkh, wvh, bvh, woh, bo[l].reshape(1, D)).reshape(N, D)

        wg_pad = jnp.pad(gr_w[l], ((0, 0), (0, 128 - G)))
        bg_pad = jnp.pad(gr_b[l].reshape(1, G), ((0, 0), (0, 128 - G)))
        we_pad = jnp.pad(er_w[l], ((0, 0), (0, 128 - E)))
        be_pad = jnp.pad(er_b[l].reshape(1, E), ((0, 0), (0, 128 - E)))
        hn2, meta, tev = pl.pallas_call(
            _router_body,
            out_shape=(jax.ShapeDtypeStruct((N, D), jnp.bfloat16),
                       jax.ShapeDtypeStruct((N, 128), f32),
                       jax.ShapeDtypeStruct((8, 128), jnp.int32)),
        )(h, ln2_g[l].reshape(1, D), ln2_b[l].reshape(1, D),
          wg_pad, bg_pad, we_pad, be_pad)

        d1 = meta[:, 0].astype(jnp.int32)
        d2c = meta[:, 1].astype(jnp.int32)
        g1 = meta[:, 2]
        g2 = meta[:, 3]
        dest = jnp.stack([d1, d2c], 1).reshape(-1)
        xg = jnp.zeros((P, D), jnp.bfloat16).at[dest].set(
            jnp.repeat(hn2, TOPK, axis=0))
        te = tev[0, :NT]

        y = pl.pallas_call(
            _ffn_body,
            grid_spec=pltpu.PrefetchScalarGridSpec(
                num_scalar_prefetch=1,
                grid=(NT,),
                in_specs=[
                    pl.BlockSpec((TILE, D), lambda t, te_r: (t, 0)),
                    pl.BlockSpec((None, D, DFF), lambda t, te_r: (te_r[t], 0, 0)),
                    pl.BlockSpec((None, 1, DFF), lambda t, te_r: (te_r[t], 0, 0)),
                    pl.BlockSpec((None, DFF, D), lambda t, te_r: (te_r[t], 0, 0)),
                    pl.BlockSpec((None, 1, D), lambda t, te_r: (te_r[t], 0, 0)),
                ],
                out_specs=pl.BlockSpec((TILE, D), lambda t, te_r: (t, 0)),
            ),
            out_shape=jax.ShapeDtypeStruct((P, D), f32),
        )(te, xg, w1[l].astype(jnp.bfloat16), b1[l].reshape(E, 1, DFF),
          w2[l].astype(jnp.bfloat16), b2[l].reshape(E, 1, D))

        if l < L - 1:
            yg = y[jnp.stack([d1, d2c], 1)]
            h = h + yg[:, 0] * g1[:, None] + yg[:, 1] * g2[:, None]
        else:
            payload = jnp.stack([jnp.stack([g1, g2], 1).reshape(-1),
                                 jnp.repeat(jnp.arange(N, dtype=f32) // S,
                                            TOPK)], 1)
            slotp = jnp.zeros((P, 2), f32).at[dest].set(payload)
            wb = jnp.where(jnp.arange(8, dtype=f32)[:, None] == slotp[None, :, 1],
                           slotp[None, :, 0] * (1.0 / S), 0.0)

    hw_pad = jnp.pad(head_w, ((0, 0), (0, 128 - NCLS)))
    bh_pad = jnp.pad(head_b.reshape(1, NCLS), ((0, 0), (0, 128 - NCLS)))
    logits = pl.pallas_call(
        _head_body,
        out_shape=jax.ShapeDtypeStruct((8, 128), f32),
    )(h, wb, y, lnf_g.reshape(1, D), lnf_b.reshape(1, D), hw_pad, bh_pad)
    return logits[:B, :NCLS]


# speedup vs baseline: 1.8952x; 1.0789x over previous
import jax
import jax.numpy as jnp
from jax import lax
from jax.experimental import pallas as pl
from jax.experimental.pallas import tpu as pltpu

B = 4; IN_CH = 80; T = 512; D = 256; DFF = 1024; G = 4; EPG = 4; E = 16
H = 4; DH = D // H; L = 2; S = 512; TOPK = 2; NCLS = 35
N = B * S
TILE = 128
P = N * TOPK + E * TILE
NT = P // TILE
KIN = 3 * IN_CH


def _gelu(v):
    return 0.5 * v * (1.0 + lax.erf(v * 0.7071067811865476))


def _layernorm(x, g, b):
    m = jnp.mean(x, axis=-1, keepdims=True)
    v = jnp.mean((x - m) ** 2, axis=-1, keepdims=True)
    return (x - m) * lax.rsqrt(v + 1e-5) * g + b


def _stem_body(x_ref, w_ref, b_ref, pos_ref, o_ref):
    y = jnp.dot(x_ref[...], w_ref[...], preferred_element_type=jnp.float32)
    o_ref[...] = _gelu(y + b_ref[...]) + pos_ref[...]


def _attn_body(h_ref, g_ref, bb_ref, wq_ref, bq_ref, wk_ref, bk_ref,
               wv_ref, bv_ref, wo_ref, bo_ref, o_ref):
    hh = h_ref[...]
    hn = _layernorm(hh, g_ref[...], bb_ref[...])
    q = jnp.dot(hn, wq_ref[...], preferred_element_type=jnp.float32) + bq_ref[...]
    k = jnp.dot(hn, wk_ref[...], preferred_element_type=jnp.float32) + bk_ref[...]
    v = jnp.dot(hn, wv_ref[...], preferred_element_type=jnp.float32) + bv_ref[...]
    acc = hh + bo_ref[...]
    for hd in range(H):
        qh = q[:, hd * DH:(hd + 1) * DH]
        kh = k[:, hd * DH:(hd + 1) * DH]
        vh = v[:, hd * DH:(hd + 1) * DH]
        s = lax.dot_general(qh, kh, (((1,), (1,)), ((), ())),
                            preferred_element_type=jnp.float32) * (DH ** -0.5)
        s = s - jnp.max(s, axis=-1, keepdims=True)
        e = jnp.exp(s)
        att = e / jnp.sum(e, axis=-1, keepdims=True)
        oh = jnp.dot(att, vh, preferred_element_type=jnp.float32)
        acc = acc + jnp.dot(oh, wo_ref[hd], preferred_element_type=jnp.float32)
    o_ref[...] = acc


def _router_body(h_ref, g_ref, b_ref, wg_ref, bg_ref, we_ref, be_ref,
                 hn_ref, meta_ref, te_ref):
    hn = _layernorm(h_ref[...], g_ref[...], b_ref[...])
    hn_ref[...] = hn.astype(jnp.bfloat16)
    gl = jnp.dot(hn, wg_ref[...], preferred_element_type=jnp.float32) + bg_ref[...]
    el = jnp.dot(hn, we_ref[...], preferred_element_type=jnp.float32) + be_ref[...]
    lane = lax.broadcasted_iota(jnp.int32, gl.shape, 1)
    neg = jnp.float32(-1e30)
    glm = jnp.where(lane < G, gl, neg)
    ge = jnp.where(lane < G, jnp.exp(glm - jnp.max(glm, -1, keepdims=True)), 0.0)
    gp = ge / jnp.sum(ge, -1, keepdims=True)
    elm = jnp.where(lane < E, el, neg)
    ee = jnp.where(lane < E, jnp.exp(elm - jnp.max(elm, -1, keepdims=True)), 0.0)
    r = lax.broadcasted_iota(jnp.int32, (128, 128), 0)
    c = lax.broadcasted_iota(jnp.int32, (128, 128), 1)
    grpsum = jnp.where((r < E) & (c < E) & ((r // EPG) == (c // EPG)), 1.0, 0.0)
    es = jnp.dot(ee, grpsum, preferred_element_type=jnp.float32,
                 precision=lax.Precision.HIGHEST)
    ep = ee / jnp.where(lane < E, es, 1.0)
    bcast = jnp.where((r < G) & (c < E) & ((c // EPG) == r), 1.0, 0.0)
    gpb = jnp.dot(gp, bcast, preferred_element_type=jnp.float32,
                  precision=lax.Precision.HIGHEST)
    comb = jnp.where(lane < E, gpb * ep, neg)
    big = jnp.int32(1 << 30)
    v1 = jnp.max(comb, -1, keepdims=True)
    a1 = jnp.min(jnp.where(comb == v1, lane, big), -1, keepdims=True)
    comb2 = jnp.where(lane == a1, neg, comb)
    v2 = jnp.max(comb2, -1, keepdims=True)
    a2 = jnp.min(jnp.where(comb2 == v2, lane, big), -1, keepdims=True)
    ssum = v1 + v2 + 1e-9
    oh1 = jnp.where((lane < E) & (lane == a1), 1.0, 0.0)
    oh2 = jnp.where((lane < E) & (lane == a2), 1.0, 0.0)
    u = oh1 + oh2
    tril = jnp.where(r > c, 1.0, 0.0)
    blocks = []
    carry = jnp.zeros((1, 128), jnp.float32)
    for cb in range(N // 128):
        blk = u[cb * 128:(cb + 1) * 128, :]
        blocks.append(jnp.dot(tril, blk, preferred_element_type=jnp.float32,
                              precision=lax.Precision.HIGHEST) + carry)
        carry = carry + jnp.sum(blk, 0, keepdims=True)
    uex = jnp.concatenate(blocks, axis=0)
    counts = carry
    pc = jnp.floor((counts + (TILE - 1)) * (1.0 / TILE)) * TILE
    sup = jnp.where((r < c) & (c < E), 1.0, 0.0)
    pstart = jnp.dot(pc, sup, preferred_element_type=jnp.float32,
                     precision=lax.Precision.HIGHEST)
    d1 = jnp.sum(oh1 * (pstart + uex), -1, keepdims=True)
    d2 = jnp.sum(oh2 * (pstart + uex + oh1), -1, keepdims=True)
    ts = (lax.broadcasted_iota(jnp.int32, (8, 128), 1) * TILE).astype(jnp.float32)
    acc = jnp.zeros((8, 128), jnp.float32)
    for e in range(E):
        acc = acc + jnp.where(ts >= pstart[0:1, e:e + 1], 1.0, 0.0)
    te_ref[...] = jnp.clip(acc - 1.0, 0.0, E - 1).astype(jnp.int32)
    meta = jnp.where(lane == 0, d1, 0.0)
    meta = meta + jnp.where(lane == 1, d2, 0.0)
    meta = meta + jnp.where(lane == 2, v1 / ssum, 0.0)
    meta = meta + jnp.where(lane == 3, v2 / ssum, 0.0)
    meta_ref[...] = meta


def _ffn_body(te_ref, x_ref, w1_ref, b1_ref, w2_ref, b2_ref, o_ref):
    h1 = _gelu(jnp.dot(x_ref[...], w1_ref[...],
                       preferred_element_type=jnp.float32) + b1_ref[...])
    o_ref[...] = jnp.dot(h1.astype(jnp.bfloat16), w2_ref[...],
                         preferred_element_type=jnp.float32) + b2_ref[...]


def _head_body(h_ref, wb_ref, y_ref, g_ref, b_ref, w_ref, bh_ref, o_ref):
    hh = h_ref[...]
    r = lax.broadcasted_iota(jnp.int32, (8, N), 0)
    c = lax.broadcasted_iota(jnp.int32, (8, N), 1)
    pool = jnp.where((c // S) == r, 1.0 / S, 0.0)
    p = jnp.dot(pool, hh, preferred_element_type=jnp.float32,
                precision=lax.Precision.HIGHEST)
    p = p + jnp.dot(wb_ref[...], y_ref[...], preferred_element_type=jnp.float32,
                    precision=lax.Precision.HIGHEST)
    pn = _layernorm(p, g_ref[...], b_ref[...])
    o_ref[...] = jnp.dot(pn, w_ref[...],
                         preferred_element_type=jnp.float32) + bh_ref[...]


def kernel(x, conv_w, conv_b, pos_emb, ln1_g, ln1_b, wq, bq, wk, bk, wv, bv,
           wo, bo, ln2_g, ln2_b, gr_w, gr_b, er_w, er_b, w1, b1, w2, b2,
           lnf_g, lnf_b, head_w, head_b):
    f32 = jnp.float32

    xt = jnp.transpose(x, (0, 2, 1))
    xp = jnp.pad(xt, ((0, 0), (1, 1), (0, 0)))
    win = jnp.concatenate([xp[:, 0:T], xp[:, 1:T + 1], xp[:, 2:T + 2]], -1)
    xi = jnp.pad(win.reshape(N, KIN), ((0, 0), (0, 256 - KIN)))
    wmat = jnp.pad(conv_w.transpose(2, 1, 0).reshape(KIN, D),
                   ((0, 256 - KIN), (0, 0)))
    pos = jnp.tile(pos_emb[:S], (B, 1))
    h = pl.pallas_call(
        _stem_body,
        out_shape=jax.ShapeDtypeStruct((N, D), f32),
    )(xi, wmat, conv_b.reshape(1, D), pos)

    for l in range(L):
        h4 = h.reshape(B, S, D)
        woh = wo[l].reshape(H, DH, D)
        h = pl.pallas_call(
            _attn_body,
            grid=(B,),
            in_specs=[
                pl.BlockSpec((None, S, D), lambda b: (b, 0, 0)),
                pl.BlockSpec((1, D), lambda b: (0, 0)),
                pl.BlockSpec((1, D), lambda b: (0, 0)),
                pl.BlockSpec((D, D), lambda b: (0, 0)),
                pl.BlockSpec((1, D), lambda b: (0, 0)),
                pl.BlockSpec((D, D), lambda b: (0, 0)),
                pl.BlockSpec((1, D), lambda b: (0, 0)),
                pl.BlockSpec((D, D), lambda b: (0, 0)),
                pl.BlockSpec((1, D), lambda b: (0, 0)),
                pl.BlockSpec((H, DH, D), lambda b: (0, 0, 0)),
                pl.BlockSpec((1, D), lambda b: (0, 0)),
            ],
            out_specs=pl.BlockSpec((None, S, D), lambda b: (b, 0, 0)),
            out_shape=jax.ShapeDtypeStruct((B, S, D), f32),
        )(h4, ln1_g[l].reshape(1, D), ln1_b[l].reshape(1, D),
          wq[l], bq[l].reshape(1, D), wk[l], bk[l].reshape(1, D),
          wv[l], bv[l].reshape(1, D), woh, bo[l].reshape(1, D)).reshape(N, D)

        wg_pad = jnp.pad(gr_w[l], ((0, 0), (0, 128 - G)))
        bg_pad = jnp.pad(gr_b[l].reshape(1, G), ((0, 0), (0, 128 - G)))
        we_pad = jnp.pad(er_w[l], ((0, 0), (0, 128 - E)))
        be_pad = jnp.pad(er_b[l].reshape(1, E), ((0, 0), (0, 128 - E)))
        hn2, meta, tev = pl.pallas_call(
            _router_body,
            out_shape=(jax.ShapeDtypeStruct((N, D), jnp.bfloat16),
                       jax.ShapeDtypeStruct((N, 128), f32),
                       jax.ShapeDtypeStruct((8, 128), jnp.int32)),
        )(h, ln2_g[l].reshape(1, D), ln2_b[l].reshape(1, D),
          wg_pad, bg_pad, we_pad, be_pad)

        d1 = meta[:, 0].astype(jnp.int32)
        d2c = meta[:, 1].astype(jnp.int32)
        g1 = meta[:, 2]
        g2 = meta[:, 3]
        dest = jnp.stack([d1, d2c], 1).reshape(-1)
        xg = jnp.zeros((P, D), jnp.bfloat16).at[dest].set(
            jnp.repeat(hn2, TOPK, axis=0))
        te = tev[0, :NT]

        y = pl.pallas_call(
            _ffn_body,
            grid_spec=pltpu.PrefetchScalarGridSpec(
                num_scalar_prefetch=1,
                grid=(NT,),
                in_specs=[
                    pl.BlockSpec((TILE, D), lambda t, te_r: (t, 0)),
                    pl.BlockSpec((None, D, DFF), lambda t, te_r: (te_r[t], 0, 0)),
                    pl.BlockSpec((None, 1, DFF), lambda t, te_r: (te_r[t], 0, 0)),
                    pl.BlockSpec((None, DFF, D), lambda t, te_r: (te_r[t], 0, 0)),
                    pl.BlockSpec((None, 1, D), lambda t, te_r: (te_r[t], 0, 0)),
                ],
                out_specs=pl.BlockSpec((TILE, D), lambda t, te_r: (t, 0)),
            ),
            out_shape=jax.ShapeDtypeStruct((P, D), f32),
        )(te, xg, w1[l].astype(jnp.bfloat16), b1[l].reshape(E, 1, DFF),
          w2[l].astype(jnp.bfloat16), b2[l].reshape(E, 1, D))

        if l < L - 1:
            yg = y[jnp.stack([d1, d2c], 1)]
            h = h + yg[:, 0] * g1[:, None] + yg[:, 1] * g2[:, None]
        else:
            payload = jnp.stack([jnp.stack([g1, g2], 1).reshape(-1),
                                 jnp.repeat(jnp.arange(N, dtype=f32) // S,
                                            TOPK)], 1)
            slotp = jnp.zeros((P, 2), f32).at[dest].set(payload)
            wb = jnp.where(jnp.arange(8, dtype=f32)[:, None] == slotp[None, :, 1],
                           slotp[None, :, 0] * (1.0 / S), 0.0)

    hw_pad = jnp.pad(head_w, ((0, 0), (0, 128 - NCLS)))
    bh_pad = jnp.pad(head_b.reshape(1, NCLS), ((0, 0), (0, 128 - NCLS)))
    logits = pl.pallas_call(
        _head_body,
        out_shape=jax.ShapeDtypeStruct((8, 128), f32),
    )(h, wb, y, lnf_g.reshape(1, D), lnf_b.reshape(1, D), hw_pad, bh_pad)
    return logits[:B, :NCLS]


# speedup vs baseline: 1.9246x; 1.0155x over previous
import jax
import jax.numpy as jnp
from jax import lax
from jax.experimental import pallas as pl
from jax.experimental.pallas import tpu as pltpu

B = 4; IN_CH = 80; T = 512; D = 256; DFF = 1024; G = 4; EPG = 4; E = 16
H = 4; DH = D // H; L = 2; S = 512; TOPK = 2; NCLS = 35
N = B * S
TILE = 128
P = N * TOPK + E * TILE
NT = P // TILE
KIN = 3 * IN_CH


def _gelu(v):
    return 0.5 * v * (1.0 + lax.erf(v * 0.7071067811865476))


def _layernorm(x, g, b):
    m = jnp.mean(x, axis=-1, keepdims=True)
    v = jnp.mean((x - m) ** 2, axis=-1, keepdims=True)
    return (x - m) * lax.rsqrt(v + 1e-5) * g + b


def _stem(x_ref, w_ref, b_ref, pos_ref):
    y = jnp.dot(x_ref[...], w_ref[...], preferred_element_type=jnp.float32)
    return _gelu(y + b_ref[...]) + pos_ref[...]


def _attn_tail(hh, g_ref, bb_ref, wq_ref, bq_ref, wk_ref, bk_ref,
               wv_ref, bv_ref, wo_ref, bo_ref, o_ref):
    hn = _layernorm(hh, g_ref[...], bb_ref[...])
    q = jnp.dot(hn, wq_ref[...], preferred_element_type=jnp.float32) + bq_ref[...]
    k = jnp.dot(hn, wk_ref[...], preferred_element_type=jnp.float32) + bk_ref[...]
    v = jnp.dot(hn, wv_ref[...], preferred_element_type=jnp.float32) + bv_ref[...]
    acc = hh + bo_ref[...]
    for hd in range(H):
        qh = q[:, hd * DH:(hd + 1) * DH]
        kh = k[:, hd * DH:(hd + 1) * DH]
        vh = v[:, hd * DH:(hd + 1) * DH]
        s = lax.dot_general(qh, kh, (((1,), (1,)), ((), ())),
                            preferred_element_type=jnp.float32) * (DH ** -0.5)
        s = s - jnp.max(s, axis=-1, keepdims=True)
        e = jnp.exp(s)
        att = e / jnp.sum(e, axis=-1, keepdims=True)
        oh = jnp.dot(att, vh, preferred_element_type=jnp.float32)
        acc = acc + jnp.dot(oh, wo_ref[hd], preferred_element_type=jnp.float32)
    o_ref[...] = acc


def _attn_body(h_ref, *rest):
    _attn_tail(h_ref[...], *rest)


def _attn0_body(xi_ref, wm_ref, cb_ref, pos_ref, *rest):
    _attn_tail(_stem(xi_ref, wm_ref, cb_ref, pos_ref), *rest)


def _router_body(h_ref, g_ref, b_ref, wg_ref, bg_ref, we_ref, be_ref,
                 hn_ref, meta_ref, te_ref):
    hn = _layernorm(h_ref[...], g_ref[...], b_ref[...])
    hn_ref[...] = hn.astype(jnp.bfloat16)
    gl = jnp.dot(hn, wg_ref[...], preferred_element_type=jnp.float32) + bg_ref[...]
    el = jnp.dot(hn, we_ref[...], preferred_element_type=jnp.float32) + be_ref[...]
    lane = lax.broadcasted_iota(jnp.int32, gl.shape, 1)
    neg = jnp.float32(-1e30)
    glm = jnp.where(lane < G, gl, neg)
    ge = jnp.where(lane < G, jnp.exp(glm - jnp.max(glm, -1, keepdims=True)), 0.0)
    gp = ge / jnp.sum(ge, -1, keepdims=True)
    elm = jnp.where(lane < E, el, neg)
    ee = jnp.where(lane < E, jnp.exp(elm - jnp.max(elm, -1, keepdims=True)), 0.0)
    r = lax.broadcasted_iota(jnp.int32, (128, 128), 0)
    c = lax.broadcasted_iota(jnp.int32, (128, 128), 1)
    grpsum = jnp.where((r < E) & (c < E) & ((r // EPG) == (c // EPG)), 1.0, 0.0)
    es = jnp.dot(ee, grpsum, preferred_element_type=jnp.float32,
                 precision=lax.Precision.HIGHEST)
    ep = ee / jnp.where(lane < E, es, 1.0)
    bcast = jnp.where((r < G) & (c < E) & ((c // EPG) == r), 1.0, 0.0)
    gpb = jnp.dot(gp, bcast, preferred_element_type=jnp.float32,
                  precision=lax.Precision.HIGHEST)
    comb = jnp.where(lane < E, gpb * ep, neg)
    big = jnp.int32(1 << 30)
    v1 = jnp.max(comb, -1, keepdims=True)
    a1 = jnp.min(jnp.where(comb == v1, lane, big), -1, keepdims=True)
    comb2 = jnp.where(lane == a1, neg, comb)
    v2 = jnp.max(comb2, -1, keepdims=True)
    a2 = jnp.min(jnp.where(comb2 == v2, lane, big), -1, keepdims=True)
    ssum = v1 + v2 + 1e-9
    oh1 = jnp.where((lane < E) & (lane == a1), 1.0, 0.0)
    oh2 = jnp.where((lane < E) & (lane == a2), 1.0, 0.0)
    u = oh1 + oh2
    tril = jnp.where(r > c, 1.0, 0.0)
    blocks = []
    carry = jnp.zeros((1, 128), jnp.float32)
    for cb in range(N // 128):
        blk = u[cb * 128:(cb + 1) * 128, :]
        blocks.append(jnp.dot(tril, blk, preferred_element_type=jnp.float32,
                              precision=lax.Precision.HIGHEST) + carry)
        carry = carry + jnp.sum(blk, 0, keepdims=True)
    uex = jnp.concatenate(blocks, axis=0)
    counts = carry
    pc = jnp.floor((counts + (TILE - 1)) * (1.0 / TILE)) * TILE
    sup = jnp.where((r < c) & (c < E), 1.0, 0.0)
    pstart = jnp.dot(pc, sup, preferred_element_type=jnp.float32,
                     precision=lax.Precision.HIGHEST)
    d1 = jnp.sum(oh1 * (pstart + uex), -1, keepdims=True)
    d2 = jnp.sum(oh2 * (pstart + uex + oh1), -1, keepdims=True)
    ts = (lax.broadcasted_iota(jnp.int32, (8, 128), 1) * TILE).astype(jnp.float32)
    acc = jnp.zeros((8, 128), jnp.float32)
    for e in range(E):
        acc = acc + jnp.where(ts >= pstart[0:1, e:e + 1], 1.0, 0.0)
    te_ref[...] = jnp.clip(acc - 1.0, 0.0, E - 1).astype(jnp.int32)
    meta = jnp.where(lane == 0, d1, 0.0)
    meta = meta + jnp.where(lane == 1, d2, 0.0)
    meta = meta + jnp.where(lane == 2, v1 / ssum, 0.0)
    meta = meta + jnp.where(lane == 3, v2 / ssum, 0.0)
    meta_ref[...] = meta


def _ffn_body(te_ref, x_ref, w1_ref, b1_ref, w2_ref, b2_ref, o_ref):
    h1 = _gelu(jnp.dot(x_ref[...], w1_ref[...],
                       preferred_element_type=jnp.float32) + b1_ref[...])
    o_ref[...] = jnp.dot(h1.astype(jnp.bfloat16), w2_ref[...],
                         preferred_element_type=jnp.float32) + b2_ref[...]


def _head_body(h_ref, wb_ref, y_ref, g_ref, b_ref, w_ref, bh_ref, o_ref):
    hh = h_ref[...]
    r = lax.broadcasted_iota(jnp.int32, (8, N), 0)
    c = lax.broadcasted_iota(jnp.int32, (8, N), 1)
    pool = jnp.where((c // S) == r, 1.0 / S, 0.0)
    p = jnp.dot(pool, hh, preferred_element_type=jnp.float32,
                precision=lax.Precision.HIGHEST)
    p = p + jnp.dot(wb_ref[...], y_ref[...], preferred_element_type=jnp.float32,
                    precision=lax.Precision.HIGHEST)
    pn = _layernorm(p, g_ref[...], b_ref[...])
    o_ref[...] = jnp.dot(pn, w_ref[...],
                         preferred_element_type=jnp.float32) + bh_ref[...]


def kernel(x, conv_w, conv_b, pos_emb, ln1_g, ln1_b, wq, bq, wk, bk, wv, bv,
           wo, bo, ln2_g, ln2_b, gr_w, gr_b, er_w, er_b, w1, b1, w2, b2,
           lnf_g, lnf_b, head_w, head_b):
    f32 = jnp.float32

    xt = jnp.transpose(x, (0, 2, 1))
    xp = jnp.pad(xt, ((0, 0), (1, 1), (0, 0)))
    win = jnp.concatenate([xp[:, 0:T], xp[:, 1:T + 1], xp[:, 2:T + 2]], -1)
    xi = jnp.pad(win.reshape(N, KIN), ((0, 0), (0, 256 - KIN)))
    wmat = jnp.pad(conv_w.transpose(2, 1, 0).reshape(KIN, D),
                   ((0, 256 - KIN), (0, 0)))
    pos = pos_emb[:S]

    for l in range(L):
        woh = wo[l].reshape(H, DH, D)
        wspecs = [
            pl.BlockSpec((1, D), lambda b: (0, 0)),
            pl.BlockSpec((1, D), lambda b: (0, 0)),
            pl.BlockSpec((D, D), lambda b: (0, 0)),
            pl.BlockSpec((1, D), lambda b: (0, 0)),
            pl.BlockSpec((D, D), lambda b: (0, 0)),
            pl.BlockSpec((1, D), lambda b: (0, 0)),
            pl.BlockSpec((D, D), lambda b: (0, 0)),
            pl.BlockSpec((1, D), lambda b: (0, 0)),
            pl.BlockSpec((H, DH, D), lambda b: (0, 0, 0)),
            pl.BlockSpec((1, D), lambda b: (0, 0)),
        ]
        wargs = (ln1_g[l].reshape(1, D), ln1_b[l].reshape(1, D),
                 wq[l], bq[l].reshape(1, D), wk[l], bk[l].reshape(1, D),
                 wv[l], bv[l].reshape(1, D), woh, bo[l].reshape(1, D))
        if l == 0:
            h = pl.pallas_call(
                _attn0_body,
                grid=(B,),
                in_specs=[
                    pl.BlockSpec((None, S, 256), lambda b: (b, 0, 0)),
                    pl.BlockSpec((256, D), lambda b: (0, 0)),
                    pl.BlockSpec((1, D), lambda b: (0, 0)),
                    pl.BlockSpec((S, D), lambda b: (0, 0)),
                ] + wspecs,
                out_specs=pl.BlockSpec((None, S, D), lambda b: (b, 0, 0)),
                out_shape=jax.ShapeDtypeStruct((B, S, D), f32),
            )(xi.reshape(B, S, 256), wmat, conv_b.reshape(1, D), pos,
              *wargs).reshape(N, D)
        else:
            h = pl.pallas_call(
                _attn_body,
                grid=(B,),
                in_specs=[pl.BlockSpec((None, S, D), lambda b: (b, 0, 0))]
                + wspecs,
                out_specs=pl.BlockSpec((None, S, D), lambda b: (b, 0, 0)),
                out_shape=jax.ShapeDtypeStruct((B, S, D), f32),
            )(h.reshape(B, S, D), *wargs).reshape(N, D)

        wg_pad = jnp.pad(gr_w[l], ((0, 0), (0, 128 - G)))
        bg_pad = jnp.pad(gr_b[l].reshape(1, G), ((0, 0), (0, 128 - G)))
        we_pad = jnp.pad(er_w[l], ((0, 0), (0, 128 - E)))
        be_pad = jnp.pad(er_b[l].reshape(1, E), ((0, 0), (0, 128 - E)))
        hn2, meta, tev = pl.pallas_call(
            _router_body,
            out_shape=(jax.ShapeDtypeStruct((N, D), jnp.bfloat16),
                       jax.ShapeDtypeStruct((N, 128), f32),
                       jax.ShapeDtypeStruct((8, 128), jnp.int32)),
        )(h, ln2_g[l].reshape(1, D), ln2_b[l].reshape(1, D),
          wg_pad, bg_pad, we_pad, be_pad)

        d1 = meta[:, 0].astype(jnp.int32)
        d2c = meta[:, 1].astype(jnp.int32)
        g1 = meta[:, 2]
        g2 = meta[:, 3]
        dest = jnp.stack([d1, d2c], 1).reshape(-1)
        xg = jnp.zeros((P, D), jnp.bfloat16).at[dest].set(
            jnp.repeat(hn2, TOPK, axis=0))
        te = tev[0, :NT]

        y = pl.pallas_call(
            _ffn_body,
            grid_spec=pltpu.PrefetchScalarGridSpec(
                num_scalar_prefetch=1,
                grid=(NT,),
                in_specs=[
                    pl.BlockSpec((TILE, D), lambda t, te_r: (t, 0)),
                    pl.BlockSpec((None, D, DFF), lambda t, te_r: (te_r[t], 0, 0)),
                    pl.BlockSpec((None, 1, DFF), lambda t, te_r: (te_r[t], 0, 0)),
                    pl.BlockSpec((None, DFF, D), lambda t, te_r: (te_r[t], 0, 0)),
                    pl.BlockSpec((None, 1, D), lambda t, te_r: (te_r[t], 0, 0)),
                ],
                out_specs=pl.BlockSpec((TILE, D), lambda t, te_r: (t, 0)),
            ),
            out_shape=jax.ShapeDtypeStruct((P, D), f32),
        )(te, xg, w1[l].astype(jnp.bfloat16), b1[l].reshape(E, 1, DFF),
          w2[l].astype(jnp.bfloat16), b2[l].reshape(E, 1, D))

        if l < L - 1:
            yg = y[jnp.stack([d1, d2c], 1)]
            h = h + yg[:, 0] * g1[:, None] + yg[:, 1] * g2[:, None]
        else:
            payload = jnp.stack([jnp.stack([g1, g2], 1).reshape(-1),
                                 jnp.repeat(jnp.arange(N, dtype=f32) // S,
                                            TOPK)], 1)
            slotp = jnp.zeros((P, 2), f32).at[dest].set(payload)
            wb = jnp.where(jnp.arange(8, dtype=f32)[:, None] == slotp[None, :, 1],
                           slotp[None, :, 0] * (1.0 / S), 0.0)

    hw_pad = jnp.pad(head_w, ((0, 0), (0, 128 - NCLS)))
    bh_pad = jnp.pad(head_b.reshape(1, NCLS), ((0, 0), (0, 128 - NCLS)))
    logits = pl.pallas_call(
        _head_body,
        out_shape=jax.ShapeDtypeStruct((8, 128), f32),
    )(h, wb, y, lnf_g.reshape(1, D), lnf_b.reshape(1, D), hw_pad, bh_pad)
    return logits[:B, :NCLS]
